# baseline (speedup 1.0000x reference)
import functools

import jax
import jax.numpy as jnp
from jax import lax
from jax.experimental import pallas as pl
from jax.experimental.pallas import tpu as pltpu
from jax.experimental.pallas import tpu_sc as plsc

N_NODES = 10000
D_FEAT = 128
N_EDGES = 320000
LANES = 16
NUM_CORES = 2
NUM_SUBCORES = 16
NUM_WORKERS = NUM_CORES * NUM_SUBCORES
EDGES_PER_WORKER = N_EDGES // NUM_WORKERS
EDGE_WIN = EDGES_PER_WORKER + 240
UNROLL = 5
EDGE_UNROLL = 25
ROWS_PER_TILE = 640
ROW_CHUNK = 160
N_CHUNKS = ROWS_PER_TILE // ROW_CHUNK


def _sc_edge_kernel(h_hbm, edges_hbm, sum_out, cnt_out, pos_out,
                    pos_v, edges_v, idx_v, sum_v, cnt_v, pos_chunk, sems):
    cid = lax.axis_index("c")
    sid = lax.axis_index("s")
    wid = sid * NUM_CORES + cid

    base = wid * EDGES_PER_WORKER
    wstart = jnp.minimum((base // 128) * 128, N_EDGES - EDGE_WIN)
    wstart = pl.multiple_of(wstart, 128)
    off = base - wstart
    cp_edges = pltpu.make_async_copy(
        edges_hbm.at[:, pl.ds(wstart, EDGE_WIN)],
        edges_v, sems.at[2])
    cp_edges.start()

    rstart = jnp.minimum(sid * ROWS_PER_TILE, N_NODES - ROWS_PER_TILE)
    rstart = pl.multiple_of(rstart, 8)
    iota = lax.iota(jnp.int32, LANES)

    @plsc.parallel_loop(0, ROWS_PER_TILE // LANES, unroll=UNROLL)
    def _mkidx(i):
        idx_v[pl.ds(i * LANES, LANES)] = (rstart + i * LANES + iota) * D_FEAT

    gathers = [
        pltpu.make_async_copy(
            h_hbm.at[idx_v.at[pl.ds(k * 128, 128)]],
            pos_chunk.at[pl.ds(k * 128, 128)], sems.at[0])
        for k in range(ROWS_PER_TILE // 128)
    ]
    for g in gathers:
        g.start()

    zeros = jnp.zeros((LANES,), jnp.float32)

    @plsc.parallel_loop(0, N_NODES // LANES, unroll=UNROLL)
    def _zero(i):
        sl = pl.ds(i * LANES, LANES)
        sum_v[sl] = zeros
        cnt_v[sl] = zeros

    for g in gathers:
        g.wait()

    pltpu.sync_copy(pos_chunk, pos_out.at[pl.ds(rstart, ROWS_PER_TILE)])
    plsc.subcore_barrier()
    pltpu.sync_copy(pos_out, pos_v)

    cp_edges.wait()
    ones = jnp.ones((LANES,), jnp.float32)

    row0 = jnp.zeros((LANES,), jnp.int32)
    row1 = jnp.ones((LANES,), jnp.int32)

    @plsc.parallel_loop(0, EDGES_PER_WORKER // LANES, unroll=EDGE_UNROLL)
    def _edges(i):
        cols = iota + (off + i * LANES)
        s_idx = plsc.load_gather(edges_v, [row0, cols])
        d_idx = plsc.load_gather(edges_v, [row1, cols])
        a = plsc.load_gather(pos_v, [s_idx])
        b = plsc.load_gather(pos_v, [d_idx])
        dist = jnp.abs(a - b)
        plsc.addupdate_scatter(sum_v, [d_idx], dist)
        plsc.addupdate_scatter(cnt_v, [d_idx], ones)

    pltpu.sync_copy(sum_v, sum_out.at[wid])
    pltpu.sync_copy(cnt_v, cnt_out.at[wid])


@functools.partial(
    pl.kernel,
    out_type=(
        jax.ShapeDtypeStruct((NUM_WORKERS, N_NODES), jnp.float32),
        jax.ShapeDtypeStruct((NUM_WORKERS, N_NODES), jnp.float32),
        jax.ShapeDtypeStruct((N_NODES,), jnp.float32),
    ),
    mesh=plsc.VectorSubcoreMesh(core_axis_name="c", subcore_axis_name="s"),
    compiler_params=pltpu.CompilerParams(needs_layout_passes=False),
    scratch_types=[
        pltpu.VMEM((N_NODES,), jnp.float32),
        pltpu.VMEM((NUM_CORES, EDGE_WIN), jnp.int32),
        pltpu.VMEM((ROWS_PER_TILE,), jnp.int32),
        pltpu.VMEM((N_NODES,), jnp.float32),
        pltpu.VMEM((N_NODES,), jnp.float32),
        pltpu.VMEM((ROWS_PER_TILE,), jnp.float32),
        pltpu.SemaphoreType.DMA((3,)),
    ],
)
def _sc_partials(h_hbm, edges_hbm, sum_out, cnt_out, pos_out,
                 pos_v, edges_v, idx_v, sum_v, cnt_v, pos_chunk, sems):
    _sc_edge_kernel(h_hbm, edges_hbm, sum_out, cnt_out, pos_out,
                    pos_v, edges_v, idx_v, sum_v, cnt_v, pos_chunk, sems)


def _tc_combine_kernel(sums_ref, cnts_ref, mean_ref):
    s = jnp.sum(sums_ref[...], axis=0)
    c = jnp.sum(cnts_ref[...], axis=0)
    mean_ref[...] = s / jnp.maximum(c, 1.0)


COMBINE_BLK = 1024
COMBINE_GRID = 10


def kernel(h, edge_index):
    edge_index = edge_index.astype(jnp.int32)
    h_flat = h.reshape(N_NODES * D_FEAT)

    sums, cnts, pos = _sc_partials(h_flat, edge_index)

    mean = pl.pallas_call(
        _tc_combine_kernel,
        grid=(COMBINE_GRID,),
        in_specs=[
            pl.BlockSpec((NUM_WORKERS, COMBINE_BLK), lambda i: (0, i)),
            pl.BlockSpec((NUM_WORKERS, COMBINE_BLK), lambda i: (0, i)),
        ],
        out_specs=pl.BlockSpec((COMBINE_BLK,), lambda i: (i,)),
        out_shape=jax.ShapeDtypeStruct((N_NODES,), jnp.float32),
    )(sums, cnts)
    return jnp.stack([pos, mean], axis=1)


# speedup vs baseline: 1.0310x; 1.0310x over previous
import functools

import jax
import jax.numpy as jnp
from jax import lax
from jax.experimental import pallas as pl
from jax.experimental.pallas import tpu as pltpu
from jax.experimental.pallas import tpu_sc as plsc

N_NODES = 10000
D_FEAT = 128
N_EDGES = 320000
LANES = 16
NUM_CORES = 2
NUM_SUBCORES = 16
NUM_WORKERS = NUM_CORES * NUM_SUBCORES
EDGES_PER_WORKER = N_EDGES // NUM_WORKERS
EDGE_WIN = EDGES_PER_WORKER + 240
UNROLL = 5
EDGE_UNROLL = 5
ROWS_PER_TILE = 640
ROW_CHUNK = 160
N_CHUNKS = ROWS_PER_TILE // ROW_CHUNK


def _sc_edge_kernel(h_hbm, edges_hbm, sum_out, cnt_out, pos_out,
                    pos_v, edges_v, idx_v, sum_v, cnt_v, pos_chunk, sems):
    cid = lax.axis_index("c")
    sid = lax.axis_index("s")
    wid = sid * NUM_CORES + cid

    base = wid * EDGES_PER_WORKER
    wstart = jnp.minimum((base // 128) * 128, N_EDGES - EDGE_WIN)
    wstart = pl.multiple_of(wstart, 128)
    off = base - wstart
    cp_edges = pltpu.make_async_copy(
        edges_hbm.at[:, pl.ds(wstart, EDGE_WIN)],
        edges_v, sems.at[2])
    cp_edges.start()

    rstart = jnp.minimum(sid * ROWS_PER_TILE, N_NODES - ROWS_PER_TILE)
    rstart = pl.multiple_of(rstart, 8)
    iota = lax.iota(jnp.int32, LANES)

    @plsc.parallel_loop(0, ROWS_PER_TILE // LANES, unroll=UNROLL)
    def _mkidx(i):
        idx_v[pl.ds(i * LANES, LANES)] = (rstart + i * LANES + iota) * D_FEAT

    gathers = [
        pltpu.make_async_copy(
            h_hbm.at[idx_v.at[pl.ds(k * 128, 128)]],
            pos_chunk.at[pl.ds(k * 128, 128)], sems.at[0])
        for k in range(ROWS_PER_TILE // 128)
    ]
    for g in gathers:
        g.start()

    zeros = jnp.zeros((LANES,), jnp.float32)

    @plsc.parallel_loop(0, N_NODES // LANES, unroll=UNROLL)
    def _zero(i):
        sl = pl.ds(i * LANES, LANES)
        sum_v[sl] = zeros
        cnt_v[sl] = zeros

    for g in gathers:
        g.wait()

    pltpu.sync_copy(pos_chunk, pos_out.at[pl.ds(rstart, ROWS_PER_TILE)])
    plsc.subcore_barrier()
    pltpu.sync_copy(pos_out, pos_v)

    cp_edges.wait()
    ones = jnp.ones((LANES,), jnp.float32)

    row0 = jnp.zeros((LANES,), jnp.int32)
    row1 = jnp.ones((LANES,), jnp.int32)

    @plsc.parallel_loop(0, EDGES_PER_WORKER // LANES, unroll=EDGE_UNROLL)
    def _edges(i):
        cols = iota + (off + i * LANES)
        s_idx = plsc.load_gather(edges_v, [row0, cols])
        d_idx = plsc.load_gather(edges_v, [row1, cols])
        a = plsc.load_gather(pos_v, [s_idx])
        b = plsc.load_gather(pos_v, [d_idx])
        dist = jnp.abs(a - b)
        plsc.addupdate_scatter(sum_v, [d_idx], dist)
        plsc.addupdate_scatter(cnt_v, [d_idx], ones)

    pltpu.sync_copy(sum_v, sum_out.at[wid])
    pltpu.sync_copy(cnt_v, cnt_out.at[wid])


@functools.partial(
    pl.kernel,
    out_type=(
        jax.ShapeDtypeStruct((NUM_WORKERS, N_NODES), jnp.float32),
        jax.ShapeDtypeStruct((NUM_WORKERS, N_NODES), jnp.float32),
        jax.ShapeDtypeStruct((N_NODES,), jnp.float32),
    ),
    mesh=plsc.VectorSubcoreMesh(core_axis_name="c", subcore_axis_name="s"),
    compiler_params=pltpu.CompilerParams(needs_layout_passes=False),
    scratch_types=[
        pltpu.VMEM((N_NODES,), jnp.float32),
        pltpu.VMEM((NUM_CORES, EDGE_WIN), jnp.int32),
        pltpu.VMEM((ROWS_PER_TILE,), jnp.int32),
        pltpu.VMEM((N_NODES,), jnp.float32),
        pltpu.VMEM((N_NODES,), jnp.float32),
        pltpu.VMEM((ROWS_PER_TILE,), jnp.float32),
        pltpu.SemaphoreType.DMA((3,)),
    ],
)
def _sc_partials(h_hbm, edges_hbm, sum_out, cnt_out, pos_out,
                 pos_v, edges_v, idx_v, sum_v, cnt_v, pos_chunk, sems):
    _sc_edge_kernel(h_hbm, edges_hbm, sum_out, cnt_out, pos_out,
                    pos_v, edges_v, idx_v, sum_v, cnt_v, pos_chunk, sems)


def _tc_combine_kernel(sums_ref, cnts_ref, mean_ref):
    s = jnp.sum(sums_ref[...], axis=0)
    c = jnp.sum(cnts_ref[...], axis=0)
    mean_ref[...] = s / jnp.maximum(c, 1.0)


COMBINE_BLK = 1024
COMBINE_GRID = 10


def kernel(h, edge_index):
    edge_index = edge_index.astype(jnp.int32)
    h_flat = h.reshape(N_NODES * D_FEAT)

    sums, cnts, pos = _sc_partials(h_flat, edge_index)

    mean = pl.pallas_call(
        _tc_combine_kernel,
        grid=(COMBINE_GRID,),
        in_specs=[
            pl.BlockSpec((NUM_WORKERS, COMBINE_BLK), lambda i: (0, i)),
            pl.BlockSpec((NUM_WORKERS, COMBINE_BLK), lambda i: (0, i)),
        ],
        out_specs=pl.BlockSpec((COMBINE_BLK,), lambda i: (i,)),
        out_shape=jax.ShapeDtypeStruct((N_NODES,), jnp.float32),
    )(sums, cnts)
    return jnp.stack([pos, mean], axis=1)


# speedup vs baseline: 1.0902x; 1.0574x over previous
import functools

import jax
import jax.numpy as jnp
from jax import lax
from jax.experimental import pallas as pl
from jax.experimental.pallas import tpu as pltpu
from jax.experimental.pallas import tpu_sc as plsc

N_NODES = 10000
D_FEAT = 128
N_EDGES = 320000
LANES = 16
NUM_CORES = 2
NUM_SUBCORES = 16
NUM_WORKERS = NUM_CORES * NUM_SUBCORES
EDGES_PER_WORKER = N_EDGES // NUM_WORKERS
EDGE_WIN = EDGES_PER_WORKER + 240
UNROLL = 5
EDGE_UNROLL = 10
ROWS_PER_TILE = 640
ROW_CHUNK = 160
N_CHUNKS = ROWS_PER_TILE // ROW_CHUNK


def _sc_edge_kernel(h_hbm, edges_hbm, sum_out, cnt_out, pos_out,
                    pos_v, edges_v, idx_v, sum_v, cnt_v, pos_chunk, sems):
    cid = lax.axis_index("c")
    sid = lax.axis_index("s")
    wid = sid * NUM_CORES + cid

    base = wid * EDGES_PER_WORKER
    wstart = jnp.minimum((base // 128) * 128, N_EDGES - EDGE_WIN)
    wstart = pl.multiple_of(wstart, 128)
    off = base - wstart
    cp_edges = pltpu.make_async_copy(
        edges_hbm.at[:, pl.ds(wstart, EDGE_WIN)],
        edges_v, sems.at[2])
    cp_edges.start()

    rstart = jnp.minimum(sid * ROWS_PER_TILE, N_NODES - ROWS_PER_TILE)
    rstart = pl.multiple_of(rstart, 8)
    iota = lax.iota(jnp.int32, LANES)

    @plsc.parallel_loop(0, ROWS_PER_TILE // LANES, unroll=UNROLL)
    def _mkidx(i):
        idx_v[pl.ds(i * LANES, LANES)] = (rstart + i * LANES + iota) * D_FEAT

    gathers = [
        pltpu.make_async_copy(
            h_hbm.at[idx_v.at[pl.ds(k * 128, 128)]],
            pos_chunk.at[pl.ds(k * 128, 128)], sems.at[0])
        for k in range(ROWS_PER_TILE // 128)
    ]
    for g in gathers:
        g.start()

    zeros = jnp.zeros((LANES,), jnp.float32)

    @plsc.parallel_loop(0, N_NODES // LANES, unroll=UNROLL)
    def _zero(i):
        sl = pl.ds(i * LANES, LANES)
        sum_v[sl] = zeros
        cnt_v[sl] = zeros

    for g in gathers:
        g.wait()

    pltpu.sync_copy(pos_chunk, pos_out.at[pl.ds(rstart, ROWS_PER_TILE)])
    plsc.subcore_barrier()
    pltpu.sync_copy(pos_out, pos_v)

    cp_edges.wait()
    ones = jnp.ones((LANES,), jnp.float32)

    row0 = jnp.zeros((LANES,), jnp.int32)
    row1 = jnp.ones((LANES,), jnp.int32)

    @plsc.parallel_loop(0, EDGES_PER_WORKER // LANES, unroll=EDGE_UNROLL)
    def _edges(i):
        cols = iota + (off + i * LANES)
        s_idx = plsc.load_gather(edges_v, [row0, cols])
        d_idx = plsc.load_gather(edges_v, [row1, cols])
        a = plsc.load_gather(pos_v, [s_idx])
        b = plsc.load_gather(pos_v, [d_idx])
        dist = jnp.abs(a - b)
        plsc.addupdate_scatter(sum_v, [d_idx], dist)
        plsc.addupdate_scatter(cnt_v, [d_idx], ones)

    pltpu.sync_copy(sum_v, sum_out.at[wid])
    pltpu.sync_copy(cnt_v, cnt_out.at[wid])


@functools.partial(
    pl.kernel,
    out_type=(
        jax.ShapeDtypeStruct((NUM_WORKERS, N_NODES), jnp.float32),
        jax.ShapeDtypeStruct((NUM_WORKERS, N_NODES), jnp.float32),
        jax.ShapeDtypeStruct((N_NODES,), jnp.float32),
    ),
    mesh=plsc.VectorSubcoreMesh(core_axis_name="c", subcore_axis_name="s"),
    compiler_params=pltpu.CompilerParams(needs_layout_passes=False),
    scratch_types=[
        pltpu.VMEM((N_NODES,), jnp.float32),
        pltpu.VMEM((NUM_CORES, EDGE_WIN), jnp.int32),
        pltpu.VMEM((ROWS_PER_TILE,), jnp.int32),
        pltpu.VMEM((N_NODES,), jnp.float32),
        pltpu.VMEM((N_NODES,), jnp.float32),
        pltpu.VMEM((ROWS_PER_TILE,), jnp.float32),
        pltpu.SemaphoreType.DMA((3,)),
    ],
)
def _sc_partials(h_hbm, edges_hbm, sum_out, cnt_out, pos_out,
                 pos_v, edges_v, idx_v, sum_v, cnt_v, pos_chunk, sems):
    _sc_edge_kernel(h_hbm, edges_hbm, sum_out, cnt_out, pos_out,
                    pos_v, edges_v, idx_v, sum_v, cnt_v, pos_chunk, sems)


def _tc_combine_kernel(sums_ref, cnts_ref, mean_ref):
    s = jnp.sum(sums_ref[...], axis=0)
    c = jnp.sum(cnts_ref[...], axis=0)
    mean_ref[...] = s / jnp.maximum(c, 1.0)


COMBINE_BLK = 2048
COMBINE_GRID = 5


def kernel(h, edge_index):
    edge_index = edge_index.astype(jnp.int32)
    h_flat = h.reshape(N_NODES * D_FEAT)

    sums, cnts, pos = _sc_partials(h_flat, edge_index)

    mean = pl.pallas_call(
        _tc_combine_kernel,
        grid=(COMBINE_GRID,),
        in_specs=[
            pl.BlockSpec((NUM_WORKERS, COMBINE_BLK), lambda i: (0, i)),
            pl.BlockSpec((NUM_WORKERS, COMBINE_BLK), lambda i: (0, i)),
        ],
        out_specs=pl.BlockSpec((COMBINE_BLK,), lambda i: (i,)),
        out_shape=jax.ShapeDtypeStruct((N_NODES,), jnp.float32),
    )(sums, cnts)
    return jnp.stack([pos, mean], axis=1)


# speedup vs baseline: 1.1044x; 1.0130x over previous
import functools

import jax
import jax.numpy as jnp
from jax import lax
from jax.experimental import pallas as pl
from jax.experimental.pallas import tpu as pltpu
from jax.experimental.pallas import tpu_sc as plsc

N_NODES = 10000
D_FEAT = 128
N_EDGES = 320000
LANES = 16
NUM_CORES = 2
NUM_SUBCORES = 16
NUM_WORKERS = NUM_CORES * NUM_SUBCORES
EDGES_PER_WORKER = N_EDGES // NUM_WORKERS
EDGE_WIN = EDGES_PER_WORKER + 240
UNROLL = 5
EDGE_UNROLL = 5
ROWS_PER_TILE = 640
ROW_CHUNK = 160
N_CHUNKS = ROWS_PER_TILE // ROW_CHUNK


def _sc_edge_kernel(h_hbm, edges_hbm, sum_out, cnt_out, pos_out,
                    pos_v, edges_v, idx_v, sum_v, cnt_v, pos_chunk, sems):
    cid = lax.axis_index("c")
    sid = lax.axis_index("s")
    wid = sid * NUM_CORES + cid

    base = wid * EDGES_PER_WORKER
    wstart = jnp.minimum((base // 128) * 128, N_EDGES - EDGE_WIN)
    wstart = pl.multiple_of(wstart, 128)
    off = base - wstart
    cp_edges = pltpu.make_async_copy(
        edges_hbm.at[:, pl.ds(wstart, EDGE_WIN)],
        edges_v, sems.at[2])
    cp_edges.start()

    rstart = jnp.minimum(sid * ROWS_PER_TILE, N_NODES - ROWS_PER_TILE)
    rstart = pl.multiple_of(rstart, 8)
    iota = lax.iota(jnp.int32, LANES)

    @plsc.parallel_loop(0, ROWS_PER_TILE // LANES, unroll=UNROLL)
    def _mkidx(i):
        idx_v[pl.ds(i * LANES, LANES)] = (rstart + i * LANES + iota) * D_FEAT

    gathers = [
        pltpu.make_async_copy(
            h_hbm.at[idx_v.at[pl.ds(k * 128, 128)]],
            pos_chunk.at[pl.ds(k * 128, 128)], sems.at[0])
        for k in range(ROWS_PER_TILE // 128)
    ]
    for g in gathers:
        g.start()

    zeros = jnp.zeros((LANES,), jnp.float32)

    @plsc.parallel_loop(0, N_NODES // LANES, unroll=UNROLL)
    def _zero(i):
        sl = pl.ds(i * LANES, LANES)
        sum_v[sl] = zeros
        cnt_v[sl] = zeros

    for g in gathers:
        g.wait()

    pltpu.sync_copy(pos_chunk, pos_out.at[pl.ds(rstart, ROWS_PER_TILE)])
    plsc.subcore_barrier()
    pltpu.sync_copy(pos_out, pos_v)

    cp_edges.wait()
    ones = jnp.ones((LANES,), jnp.float32)

    row0 = jnp.zeros((LANES,), jnp.int32)
    row1 = jnp.ones((LANES,), jnp.int32)

    @plsc.parallel_loop(0, EDGES_PER_WORKER // LANES, unroll=EDGE_UNROLL)
    def _edges(i):
        sl = pl.ds(off + i * LANES, LANES)
        s_idx = edges_v[0, sl]
        d_idx = edges_v[1, sl]
        a = plsc.load_gather(pos_v, [s_idx])
        b = plsc.load_gather(pos_v, [d_idx])
        dist = jnp.abs(a - b)
        plsc.addupdate_scatter(sum_v, [d_idx], dist)
        plsc.addupdate_scatter(cnt_v, [d_idx], ones)

    pltpu.sync_copy(sum_v, sum_out.at[wid])
    pltpu.sync_copy(cnt_v, cnt_out.at[wid])


@functools.partial(
    pl.kernel,
    out_type=(
        jax.ShapeDtypeStruct((NUM_WORKERS, N_NODES), jnp.float32),
        jax.ShapeDtypeStruct((NUM_WORKERS, N_NODES), jnp.float32),
        jax.ShapeDtypeStruct((N_NODES,), jnp.float32),
    ),
    mesh=plsc.VectorSubcoreMesh(core_axis_name="c", subcore_axis_name="s"),
    compiler_params=pltpu.CompilerParams(needs_layout_passes=False),
    scratch_types=[
        pltpu.VMEM((N_NODES,), jnp.float32),
        pltpu.VMEM((NUM_CORES, EDGE_WIN), jnp.int32),
        pltpu.VMEM((ROWS_PER_TILE,), jnp.int32),
        pltpu.VMEM((N_NODES,), jnp.float32),
        pltpu.VMEM((N_NODES,), jnp.float32),
        pltpu.VMEM((ROWS_PER_TILE,), jnp.float32),
        pltpu.SemaphoreType.DMA((3,)),
    ],
)
def _sc_partials(h_hbm, edges_hbm, sum_out, cnt_out, pos_out,
                 pos_v, edges_v, idx_v, sum_v, cnt_v, pos_chunk, sems):
    _sc_edge_kernel(h_hbm, edges_hbm, sum_out, cnt_out, pos_out,
                    pos_v, edges_v, idx_v, sum_v, cnt_v, pos_chunk, sems)


def _tc_combine_kernel(sums_ref, cnts_ref, mean_ref):
    s = jnp.sum(sums_ref[...], axis=0)
    c = jnp.sum(cnts_ref[...], axis=0)
    mean_ref[...] = s / jnp.maximum(c, 1.0)


COMBINE_BLK = 2048
COMBINE_GRID = 5


def kernel(h, edge_index):
    edge_index = edge_index.astype(jnp.int32)
    h_flat = h.reshape(N_NODES * D_FEAT)

    sums, cnts, pos = _sc_partials(h_flat, edge_index)

    mean = pl.pallas_call(
        _tc_combine_kernel,
        grid=(COMBINE_GRID,),
        in_specs=[
            pl.BlockSpec((NUM_WORKERS, COMBINE_BLK), lambda i: (0, i)),
            pl.BlockSpec((NUM_WORKERS, COMBINE_BLK), lambda i: (0, i)),
        ],
        out_specs=pl.BlockSpec((COMBINE_BLK,), lambda i: (i,)),
        out_shape=jax.ShapeDtypeStruct((N_NODES,), jnp.float32),
    )(sums, cnts)
    return jnp.stack([pos, mean], axis=1)
